# Initial kernel scaffold; baseline (speedup 1.0000x reference)
import jax
import jax.numpy as jnp
from jax.experimental import pallas as pl


def kernel(user_emb, item_emb, edge_index, S_init):
    raise NotImplementedError("write your pallas kernel here")


# speedup vs baseline: 2.6403x; 2.6403x over previous
import jax
import jax.numpy as jnp
from jax.experimental import pallas as pl

NU = 25000
NI = 25000
D = 128
F = 4
DK = D // F
E = 800000


def _norm_slices_body(x_ref, o_ref):
    x = x_ref[...]
    xs = x.reshape(x.shape[0], F, DK)
    n = jnp.sqrt(jnp.sum(xs * xs, axis=2, keepdims=True))
    o_ref[...] = (xs / jnp.maximum(n, 1e-12)).reshape(x.shape)


def _norm_slices(x):
    blk = 1000
    return pl.pallas_call(
        _norm_slices_body,
        grid=(x.shape[0] // blk,),
        in_specs=[pl.BlockSpec((blk, D), lambda i: (i, 0))],
        out_specs=pl.BlockSpec((blk, D), lambda i: (i, 0)),
        out_shape=jax.ShapeDtypeStruct(x.shape, x.dtype),
    )(x)


def _tanh_norm_body(x_ref, o_ref):
    x = x_ref[...]
    xs = x.reshape(x.shape[0], F, DK)
    n = jnp.sqrt(jnp.sum(xs * xs, axis=2, keepdims=True))
    o_ref[...] = jnp.tanh((xs / jnp.maximum(n, 1e-12)).reshape(x.shape))


def _tanh_norm(x):
    blk = 1000
    return pl.pallas_call(
        _tanh_norm_body,
        grid=(x.shape[0] // blk,),
        in_specs=[pl.BlockSpec((blk, D), lambda i: (i, 0))],
        out_specs=pl.BlockSpec((blk, D), lambda i: (i, 0)),
        out_shape=jax.ShapeDtypeStruct(x.shape, x.dtype),
    )(x)


def kernel(user_emb, item_emb, edge_index, S_init):
    u = edge_index[0]
    iloc = edge_index[1] - NU
    deg_u = jnp.zeros((NU,), jnp.float32).at[u].add(1.0)
    deg_i = jnp.zeros((NI,), jnp.float32).at[iloc].add(1.0)
    dis_u = jnp.where(deg_u > 0, 1.0 / jnp.sqrt(jnp.where(deg_u > 0, deg_u, 1.0)), 0.0)
    dis_i = jnp.where(deg_i > 0, 1.0 / jnp.sqrt(jnp.where(deg_i > 0, deg_i, 1.0)), 0.0)
    dn = dis_u[u] * dis_i[iloc]

    Y = _tanh_norm(item_emb)

    S = S_init
    out_u = out_i = None
    for t in range(2):
        P = jax.nn.softmax(S, axis=0)
        w = (P * dn[None, :]).T
        msg_i = (user_emb[u].reshape(E, F, DK) * w[:, :, None]).reshape(E, D)
        out_i = jnp.zeros((NI, D), jnp.float32).at[iloc].add(msg_i)
        msg_u = (item_emb[iloc].reshape(E, F, DK) * w[:, :, None]).reshape(E, D)
        out_u = jnp.zeros((NU, D), jnp.float32).at[u].add(msg_u)
        Zu = _norm_slices(out_u)
        sval = jnp.sum(Zu[u].reshape(E, F, DK) * Y[iloc].reshape(E, F, DK), axis=2).T
        S = P + sval
    return user_emb + out_u, item_emb + out_i, S


# speedup vs baseline: 5.1080x; 1.9346x over previous
import functools

import jax
import jax.numpy as jnp
from jax import lax
from jax.experimental import pallas as pl
from jax.experimental.pallas import tpu as pltpu
from jax.experimental.pallas import tpu_sc as plsc

NU = 25000
NI = 25000
D = 128
F = 4
DK = D // F
E = 800000
H = 64
CH = 128
NCHUNK = E // CH
NT = 16
NW = 32
ZR = 1000
NZ = NU // ZR

_MESH = plsc.VectorSubcoreMesh(core_axis_name="c", subcore_axis_name="s")
_f32 = jnp.float32
_i32 = jnp.int32


def _deg_body(users, iloc, zdeg, deg_u, deg_i, acc, idx_v, ones_v):
    c = lax.axis_index("c")
    s = lax.axis_index("s")
    for g in range(CH // 16):
        ones_v[pl.ds(g * 16, 16)] = jnp.full((16,), 1.0, _f32)

    @pl.when(s == 0)
    def _():
        pltpu.sync_copy(zdeg, acc)

    plsc.subcore_barrier()

    def run(src_hbm):
        nc = (NCHUNK - s + NT - 1) // NT

        def body(i, carry):
            e0 = (s + i * NT) * CH
            pltpu.sync_copy(src_hbm.at[pl.ds(e0, CH)], idx_v)
            pltpu.sync_copy(ones_v, acc.at[idx_v], add=True)
            return carry

        lax.fori_loop(0, nc, body, 0)

    @pl.when(c == 0)
    def _():
        run(users)

    @pl.when(c == 1)
    def _():
        run(iloc)

    plsc.subcore_barrier()

    @pl.when((s == 0) & (c == 0))
    def _():
        pltpu.sync_copy(acc, deg_u)

    @pl.when((s == 0) & (c == 1))
    def _():
        pltpu.sync_copy(acc, deg_i)


_deg_call = pl.kernel(
    _deg_body,
    out_type=[
        jax.ShapeDtypeStruct((NU,), _f32),
        jax.ShapeDtypeStruct((NI,), _f32),
    ],
    mesh=_MESH,
    compiler_params=pltpu.CompilerParams(needs_layout_passes=False, use_tc_tiling_on_sc=False),
    scratch_types=[
        pltpu.VMEM_SHARED((NU,), _f32),
        pltpu.VMEM((CH,), _i32),
        pltpu.VMEM((CH,), _f32),
    ],
)


def _dn_body(dis_u, dis_i, users, iloc, dn,
             du_v, di_v, uidx_v, iidx_v, dn_v):
    c = lax.axis_index("c")
    s = lax.axis_index("s")
    gid = c * NT + s
    pltpu.sync_copy(dis_u, du_v)
    pltpu.sync_copy(dis_i, di_v)
    nc = (NCHUNK - gid + NW - 1) // NW

    def body(i, carry):
        e0 = (gid + i * NW) * CH
        pltpu.sync_copy(users.at[pl.ds(e0, CH)], uidx_v)
        pltpu.sync_copy(iloc.at[pl.ds(e0, CH)], iidx_v)
        for g in range(CH // 16):
            sl = pl.ds(g * 16, 16)
            gu = plsc.load_gather(du_v, [uidx_v[sl]])
            gi = plsc.load_gather(di_v, [iidx_v[sl]])
            dn_v[sl] = gu * gi
        pltpu.sync_copy(dn_v, dn.at[pl.ds(e0, CH)])
        return carry

    lax.fori_loop(0, nc, body, 0)


_dn_call = pl.kernel(
    _dn_body,
    out_type=[jax.ShapeDtypeStruct((E,), _f32)],
    mesh=_MESH,
    compiler_params=pltpu.CompilerParams(needs_layout_passes=False, use_tc_tiling_on_sc=False),
    scratch_types=[
        pltpu.VMEM((NU,), _f32),
        pltpu.VMEM((NI,), _f32),
        pltpu.VMEM((CH,), _i32),
        pltpu.VMEM((CH,), _i32),
        pltpu.VMEM((CH,), _f32),
    ],
)


def _softmax_w(s_v, dn_v, w0_v, w1_v, h):
    for g in range(CH // 16):
        sl = pl.ds(g * 16, 16)
        s0 = s_v[0, sl]
        s1 = s_v[1, sl]
        s2 = s_v[2, sl]
        s3 = s_v[3, sl]
        m = jnp.maximum(jnp.maximum(s0, s1), jnp.maximum(s2, s3))
        x0 = jnp.exp(s0 - m)
        x1 = jnp.exp(s1 - m)
        x2 = jnp.exp(s2 - m)
        x3 = jnp.exp(s3 - m)
        tot = (x0 + x1) + (x2 + x3)
        r = dn_v[sl] / tot
        if h == 0:
            w0_v[sl] = x0 * r
            w1_v[sl] = x1 * r
        else:
            w0_v[sl] = x2 * r
            w1_v[sl] = x3 * r


def _conv_half(table, gidx_hbm, sidx_hbm, dn, S, out_h, h,
               acc, gidx_v, sidx_v, dn_v, s_v, w0_v, w1_v, rows_v, sem,
               zblk, s):
    def zbody(i, carry):
        z = s + i * NT
        pltpu.sync_copy(zblk, acc.at[pl.ds(z * ZR, ZR)])
        return carry

    lax.fori_loop(0, (NZ - s + NT - 1) // NT, zbody, 0)
    plsc.subcore_barrier()

    def body(i, carry):
        e0 = (s + i * NT) * CH
        pltpu.sync_copy(gidx_hbm.at[pl.ds(e0, CH)], gidx_v)
        pltpu.sync_copy(sidx_hbm.at[pl.ds(e0, CH)], sidx_v)
        pltpu.sync_copy(dn.at[pl.ds(e0, CH)], dn_v)
        for k in range(F):
            pltpu.sync_copy(S.at[k, pl.ds(e0, CH)], s_v.at[k])
        pltpu.async_copy(table.at[gidx_v], rows_v, sem).wait()
        _softmax_w(s_v, dn_v, w0_v, w1_v, h)

        def sbody(j, carry2):
            bj = jnp.broadcast_to(j, (16,)).astype(_i32)
            va = plsc.load_gather(w0_v, [bj])
            vb = plsc.load_gather(w1_v, [bj])
            rows_v[j, pl.ds(0, 16)] = rows_v[j, pl.ds(0, 16)] * va
            rows_v[j, pl.ds(16, 16)] = rows_v[j, pl.ds(16, 16)] * va
            rows_v[j, pl.ds(32, 16)] = rows_v[j, pl.ds(32, 16)] * vb
            rows_v[j, pl.ds(48, 16)] = rows_v[j, pl.ds(48, 16)] * vb
            return carry2

        lax.fori_loop(0, CH, sbody, 0)
        pltpu.sync_copy(rows_v, acc.at[sidx_v], add=True)
        return carry

    lax.fori_loop(0, (NCHUNK - s + NT - 1) // NT, body, 0)
    plsc.subcore_barrier()

    def wbody(i, carry):
        z = s + i * NT
        pltpu.sync_copy(acc.at[pl.ds(z * ZR, ZR)], out_h.at[pl.ds(z * ZR, ZR)])
        return carry

    lax.fori_loop(0, (NZ - s + NT - 1) // NT, wbody, 0)
    plsc.subcore_barrier()


def _conv_body(ue0, ue1, ie0, ie1, users, iloc, S, dn, zblk,
               ou0, ou1, oi0, oi1,
               acc, gidx_v, sidx_v, dn_v, s_v, w0_v, w1_v, rows_v, sem):
    c = lax.axis_index("c")
    s = lax.axis_index("s")

    common = (acc, gidx_v, sidx_v, dn_v, s_v, w0_v, w1_v, rows_v, sem,
              zblk, s)

    @pl.when(c == 0)
    def _():
        _conv_half(ie0, iloc, users, dn, S, ou0, 0, *common)
        _conv_half(ie1, iloc, users, dn, S, ou1, 1, *common)

    @pl.when(c == 1)
    def _():
        _conv_half(ue0, users, iloc, dn, S, oi0, 0, *common)
        _conv_half(ue1, users, iloc, dn, S, oi1, 1, *common)


_conv_call = pl.kernel(
    _conv_body,
    out_type=[
        jax.ShapeDtypeStruct((NU, H), _f32),
        jax.ShapeDtypeStruct((NU, H), _f32),
        jax.ShapeDtypeStruct((NI, H), _f32),
        jax.ShapeDtypeStruct((NI, H), _f32),
    ],
    mesh=_MESH,
    compiler_params=pltpu.CompilerParams(needs_layout_passes=False, use_tc_tiling_on_sc=False),
    scratch_types=[
        pltpu.VMEM_SHARED((NU, H), _f32),
        pltpu.VMEM((CH,), _i32),
        pltpu.VMEM((CH,), _i32),
        pltpu.VMEM((CH,), _f32),
        pltpu.VMEM((F, CH), _f32),
        pltpu.VMEM((CH,), _f32),
        pltpu.VMEM((CH,), _f32),
        pltpu.VMEM((CH, H), _f32),
        pltpu.SemaphoreType.DMA,
    ],
)


def _sval_body(z0, z1, y0, y1, users, iloc, S, S_out,
               uidx_v, iidx_v, s_v, sv_v, z0r, z1r, y0r, y1r,
               sem0, sem1, sem2, sem3):
    c = lax.axis_index("c")
    s = lax.axis_index("s")
    gid = c * NT + s
    nc = (NCHUNK - gid + NW - 1) // NW

    def body(i, carry):
        e0 = (gid + i * NW) * CH
        pltpu.sync_copy(users.at[pl.ds(e0, CH)], uidx_v)
        pltpu.sync_copy(iloc.at[pl.ds(e0, CH)], iidx_v)
        for k in range(F):
            pltpu.sync_copy(S.at[k, pl.ds(e0, CH)], s_v.at[k])
        c0 = pltpu.async_copy(z0.at[uidx_v], z0r, sem0)
        c1 = pltpu.async_copy(z1.at[uidx_v], z1r, sem1)
        c2 = pltpu.async_copy(y0.at[iidx_v], y0r, sem2)
        c3 = pltpu.async_copy(y1.at[iidx_v], y1r, sem3)
        c0.wait()
        c1.wait()
        c2.wait()
        c3.wait()

        def ebody(j, carry2):
            bj = jnp.broadcast_to(j, (16,)).astype(_i32)
            for k, (zr, yr, ko) in (
                (0, (z0r, y0r, 0)),
                (1, (z0r, y0r, 0)),
                (2, (z1r, y1r, 2)),
                (3, (z1r, y1r, 2)),
            ):
                kk = k - ko
                za = zr[j, pl.ds(kk * 32, 16)]
                zb = zr[j, pl.ds(kk * 32 + 16, 16)]
                ya = yr[j, pl.ds(kk * 32, 16)]
                yb = yr[j, pl.ds(kk * 32 + 16, 16)]
                r = jnp.sum(za * ya + zb * yb)
                bk = jnp.full((16,), k, _i32)
                plsc.store_scatter(sv_v, [bk, bj], jnp.broadcast_to(r, (16,)))
            return carry2

        lax.fori_loop(0, CH, ebody, 0)

        for g in range(CH // 16):
            sl = pl.ds(g * 16, 16)
            s0 = s_v[0, sl]
            s1 = s_v[1, sl]
            s2 = s_v[2, sl]
            s3 = s_v[3, sl]
            m = jnp.maximum(jnp.maximum(s0, s1), jnp.maximum(s2, s3))
            x0 = jnp.exp(s0 - m)
            x1 = jnp.exp(s1 - m)
            x2 = jnp.exp(s2 - m)
            x3 = jnp.exp(s3 - m)
            r = jnp.full((16,), 1.0, _f32) / ((x0 + x1) + (x2 + x3))
            s_v[0, sl] = x0 * r + sv_v[0, sl]
            s_v[1, sl] = x1 * r + sv_v[1, sl]
            s_v[2, sl] = x2 * r + sv_v[2, sl]
            s_v[3, sl] = x3 * r + sv_v[3, sl]
        for k in range(F):
            pltpu.sync_copy(s_v.at[k], S_out.at[k, pl.ds(e0, CH)])
        return carry

    lax.fori_loop(0, nc, body, 0)


_sval_call = pl.kernel(
    _sval_body,
    out_type=[jax.ShapeDtypeStruct((F, E), _f32)],
    mesh=_MESH,
    compiler_params=pltpu.CompilerParams(needs_layout_passes=False, use_tc_tiling_on_sc=False),
    scratch_types=[
        pltpu.VMEM((CH,), _i32),
        pltpu.VMEM((CH,), _i32),
        pltpu.VMEM((F, CH), _f32),
        pltpu.VMEM((F, CH), _f32),
        pltpu.VMEM((CH, H), _f32),
        pltpu.VMEM((CH, H), _f32),
        pltpu.VMEM((CH, H), _f32),
        pltpu.VMEM((CH, H), _f32),
        pltpu.SemaphoreType.DMA,
        pltpu.SemaphoreType.DMA,
        pltpu.SemaphoreType.DMA,
        pltpu.SemaphoreType.DMA,
    ],
)


def _prep_body(item_ref, degu_ref, degi_ref, y0_ref, y1_ref, du_ref, di_ref):
    x = item_ref[...]
    xs = x.reshape(-1, F, DK)
    n = jnp.sqrt(jnp.sum(xs * xs, axis=2, keepdims=True))
    y = jnp.tanh((xs / jnp.maximum(n, 1e-12)).reshape(x.shape))
    y0_ref[...] = y[:, :H]
    y1_ref[...] = y[:, H:]
    for dref, oref in ((degu_ref, du_ref), (degi_ref, di_ref)):
        dg = dref[...]
        oref[...] = jnp.where(dg > 0, 1.0 / jnp.sqrt(jnp.where(dg > 0, dg, 1.0)), 0.0)


def _prep_call(item_emb, deg_u3, deg_i3):
    blk = 1000
    nb = NI // blk
    return pl.pallas_call(
        _prep_body,
        grid=(nb,),
        in_specs=[
            pl.BlockSpec((blk, D), lambda i: (i, 0)),
            pl.BlockSpec((1, 1, blk), lambda i: (i, 0, 0)),
            pl.BlockSpec((1, 1, blk), lambda i: (i, 0, 0)),
        ],
        out_specs=[
            pl.BlockSpec((blk, H), lambda i: (i, 0)),
            pl.BlockSpec((blk, H), lambda i: (i, 0)),
            pl.BlockSpec((1, 1, blk), lambda i: (i, 0, 0)),
            pl.BlockSpec((1, 1, blk), lambda i: (i, 0, 0)),
        ],
        out_shape=[
            jax.ShapeDtypeStruct((NI, H), _f32),
            jax.ShapeDtypeStruct((NI, H), _f32),
            jax.ShapeDtypeStruct((nb, 1, blk), _f32),
            jax.ShapeDtypeStruct((nb, 1, blk), _f32),
        ],
    )(item_emb, deg_u3, deg_i3)


def _znorm_body(a_ref, b_ref, z0_ref, z1_ref):
    for src, dst in ((a_ref, z0_ref), (b_ref, z1_ref)):
        x = src[...]
        xs = x.reshape(-1, 2, DK)
        n = jnp.sqrt(jnp.sum(xs * xs, axis=2, keepdims=True))
        dst[...] = (xs / jnp.maximum(n, 1e-12)).reshape(x.shape)


def _znorm_call(ou0, ou1):
    blk = 1000
    return pl.pallas_call(
        _znorm_body,
        grid=(NU // blk,),
        in_specs=[pl.BlockSpec((blk, H), lambda i: (i, 0))] * 2,
        out_specs=[pl.BlockSpec((blk, H), lambda i: (i, 0))] * 2,
        out_shape=[jax.ShapeDtypeStruct((NU, H), _f32)] * 2,
    )(ou0, ou1)


def _final_body(emb_ref, a_ref, b_ref, o_ref):
    o_ref[...] = emb_ref[...] + jnp.concatenate([a_ref[...], b_ref[...]], axis=1)


def _final_call(emb, a, b):
    blk = 1000
    return pl.pallas_call(
        _final_body,
        grid=(emb.shape[0] // blk,),
        in_specs=[
            pl.BlockSpec((blk, D), lambda i: (i, 0)),
            pl.BlockSpec((blk, H), lambda i: (i, 0)),
            pl.BlockSpec((blk, H), lambda i: (i, 0)),
        ],
        out_specs=pl.BlockSpec((blk, D), lambda i: (i, 0)),
        out_shape=jax.ShapeDtypeStruct(emb.shape, _f32),
    )(emb, a, b)


def kernel(user_emb, item_emb, edge_index, S_init):
    users = edge_index[0]
    iloc = edge_index[1] - NU
    ue0 = user_emb[:, :H]
    ue1 = user_emb[:, H:]
    ie0 = item_emb[:, :H]
    ie1 = item_emb[:, H:]
    zdeg = jnp.zeros((NU,), _f32)
    zblk = jnp.zeros((ZR, H), _f32)

    deg_u, deg_i = _deg_call(users, iloc, zdeg)
    y0, y1, du3, di3 = _prep_call(item_emb, deg_u.reshape(25, 1, 1000),
                                  deg_i.reshape(25, 1, 1000))
    dis_u = du3.reshape(NU)
    dis_i = di3.reshape(NI)
    (dn,) = _dn_call(dis_u, dis_i, users, iloc)

    S = S_init
    ou0 = ou1 = oi0 = oi1 = None
    for _t in range(2):
        ou0, ou1, oi0, oi1 = _conv_call(ue0, ue1, ie0, ie1, users, iloc, S,
                                        dn, zblk)
        zu0, zu1 = _znorm_call(ou0, ou1)
        (S,) = _sval_call(zu0, zu1, y0, y1, users, iloc, S)

    fu = _final_call(user_emb, ou0, ou1)
    fi = _final_call(item_emb, oi0, oi1)
    return fu, fi, S


# speedup vs baseline: 12.0321x; 2.3556x over previous
import functools

import jax
import jax.numpy as jnp
from jax import lax
from jax.experimental import pallas as pl
from jax.experimental.pallas import tpu as pltpu
from jax.experimental.pallas import tpu_sc as plsc

NU = 25000
NI = 25000
D = 128
F = 4
DK = D // F
E = 800000
H = 64
CH = 128
NCHUNK = E // CH
NT = 16
NW = 32
ZR = 1000
NZ = NU // ZR
NCP = 6400
EP = NCP * CH

_MESH = plsc.VectorSubcoreMesh(core_axis_name="c", subcore_axis_name="s")
_SC_PARAMS = pltpu.CompilerParams(needs_layout_passes=False,
                                  use_tc_tiling_on_sc=False)
_f32 = jnp.float32
_i32 = jnp.int32


def _deg_body(users, iloc, zdeg, deg_u, deg_i, acc, idx_v, ones_v):
    c = lax.axis_index("c")
    s = lax.axis_index("s")
    for g in range(CH // 16):
        ones_v[pl.ds(g * 16, 16)] = jnp.full((16,), 1.0, _f32)

    @pl.when(s == 0)
    def _():
        pltpu.sync_copy(zdeg, acc)

    plsc.subcore_barrier()

    def run(src_hbm):
        nc = (NCHUNK - s + NT - 1) // NT

        def body(i, carry):
            e0 = (s + i * NT) * CH
            pltpu.sync_copy(src_hbm.at[pl.ds(e0, CH)], idx_v)
            pltpu.sync_copy(ones_v, acc.at[idx_v], add=True)
            return carry

        lax.fori_loop(0, nc, body, 0)

    @pl.when(c == 0)
    def _():
        run(users)

    @pl.when(c == 1)
    def _():
        run(iloc)

    plsc.subcore_barrier()

    @pl.when((s == 0) & (c == 0))
    def _():
        pltpu.sync_copy(acc, deg_u)

    @pl.when((s == 0) & (c == 1))
    def _():
        pltpu.sync_copy(acc, deg_i)


_deg_call = pl.kernel(
    _deg_body,
    out_type=[
        jax.ShapeDtypeStruct((NU,), _f32),
        jax.ShapeDtypeStruct((NI,), _f32),
    ],
    mesh=_MESH,
    compiler_params=pltpu.CompilerParams(needs_layout_passes=False,
                                         use_tc_tiling_on_sc=False),
    scratch_types=[
        pltpu.VMEM_SHARED((NU,), _f32),
        pltpu.VMEM((CH,), _i32),
        pltpu.VMEM((CH,), _f32),
    ],
)


def _dn_body(dis_u, dis_i, edge2, dn, du_v, di_v, eidx_v, dn_v):
    c = lax.axis_index("c")
    s = lax.axis_index("s")
    gid = c * NT + s
    pltpu.sync_copy(dis_u, du_v)
    pltpu.sync_copy(dis_i, di_v)

    def body(i, carry):
        e0 = (gid + i * NW) * CH
        live = jnp.where(e0 < E, 1.0, 0.0).astype(_f32)
        pltpu.sync_copy(edge2.at[:, pl.ds(e0, CH)], eidx_v)

        def gbody(g, carry2):
            sl = pl.ds(g * 16, 16)
            gu = plsc.load_gather(du_v, [eidx_v[0, sl]])
            gi = plsc.load_gather(di_v, [eidx_v[1, sl]])
            dn_v[sl] = gu * gi * jnp.broadcast_to(live, (16,))
            return carry2

        lax.fori_loop(0, CH // 16, gbody, 0)
        pltpu.sync_copy(dn_v, dn.at[pl.ds(e0, CH)])
        return carry

    lax.fori_loop(0, NCP // NW, body, 0)


_dn_call = pl.kernel(
    _dn_body,
    out_type=[jax.ShapeDtypeStruct((EP,), _f32)],
    mesh=_MESH,
    compiler_params=pltpu.CompilerParams(needs_layout_passes=False,
                                         use_tc_tiling_on_sc=False),
    scratch_types=[
        pltpu.VMEM((NU,), _f32),
        pltpu.VMEM((NI,), _f32),
        pltpu.VMEM((2, CH), _i32),
        pltpu.VMEM((CH,), _f32),
    ],
)


def _cin_issue(edge2, S, dn, e0, ei, sv, dv, sem):
    pltpu.async_copy(edge2.at[:, pl.ds(e0, CH)], ei, sem)
    pltpu.async_copy(S.at[:, pl.ds(e0, CH)], sv, sem)
    pltpu.async_copy(dn.at[pl.ds(e0, CH)], dv, sem)


def _cin_drain(edge2, S, dn, ei, sv, dv, sem):
    pltpu.make_async_copy(edge2.at[:, pl.ds(0, CH)], ei, sem).wait()
    pltpu.make_async_copy(S.at[:, pl.ds(0, CH)], sv, sem).wait()
    pltpu.make_async_copy(dn.at[pl.ds(0, CH)], dv, sem).wait()


def _conv_compute(sv, dv, w0_v, w1_v, rows, h):
    @plsc.parallel_loop(0, CH // 16, unroll=2)
    def _(g):
        sl = pl.ds(g * 16, 16)
        s0 = sv[0, sl]
        s1 = sv[1, sl]
        s2 = sv[2, sl]
        s3 = sv[3, sl]
        m = jnp.maximum(jnp.maximum(s0, s1), jnp.maximum(s2, s3))
        x0 = jnp.exp(s0 - m)
        x1 = jnp.exp(s1 - m)
        x2 = jnp.exp(s2 - m)
        x3 = jnp.exp(s3 - m)
        r = dv[sl] / ((x0 + x1) + (x2 + x3))
        if h == 0:
            w0_v[sl] = x0 * r
            w1_v[sl] = x1 * r
        else:
            w0_v[sl] = x2 * r
            w1_v[sl] = x3 * r

    @plsc.parallel_loop(0, CH, unroll=4)
    def _(j):
        bj = jnp.broadcast_to(j, (16,)).astype(_i32)
        va = plsc.load_gather(w0_v, [bj])
        vb = plsc.load_gather(w1_v, [bj])
        rows[j, pl.ds(0, 16)] = rows[j, pl.ds(0, 16)] * va
        rows[j, pl.ds(16, 16)] = rows[j, pl.ds(16, 16)] * va
        rows[j, pl.ds(32, 16)] = rows[j, pl.ds(32, 16)] * vb
        rows[j, pl.ds(48, 16)] = rows[j, pl.ds(48, 16)] * vb


def _conv_half(table, grow, srow, edge2, dn, S, out_h, h,
               acc, ei, sv, dv, w0_v, w1_v, rowsA, rowsB,
               sem_in, sem_gA, sem_gB, sem_sA, sem_sB, zblk, s):
    def zbody(i, carry):
        z = s + i * NT
        pltpu.sync_copy(zblk, acc.at[pl.ds(z * ZR, ZR)])
        return carry

    lax.fori_loop(0, (NZ - s + NT - 1) // NT, zbody, 0)
    plsc.subcore_barrier()

    ncs = NCP // NT
    nb = ncs // 4

    def e_of(n):
        return (s + n * NT) * CH

    for i in range(4):
        _cin_issue(edge2, S, dn, e_of(i), ei[i], sv[i], dv[i], sem_in[i])

    def body(g, carry):
        def fire(i, rows, sem_g):
            _cin_drain(edge2, S, dn, ei[i], sv[i], dv[i], sem_in[i])
            return pltpu.async_copy(table.at[ei[i].at[grow]], rows, sem_g)

        n0 = 4 * g
        g0 = fire(0, rowsA, sem_gA)
        g1 = fire(1, rowsB, sem_gB)
        g0.wait()
        _conv_compute(sv[0], dv[0], w0_v, w1_v, rowsA, h)
        s0 = pltpu.async_copy(rowsA, acc.at[ei[0].at[srow]], sem_sA, add=True)
        g1.wait()
        _conv_compute(sv[1], dv[1], w0_v, w1_v, rowsB, h)
        s1 = pltpu.async_copy(rowsB, acc.at[ei[1].at[srow]], sem_sB, add=True)
        s0.wait()
        g2 = fire(2, rowsA, sem_gA)
        s1.wait()
        g3 = fire(3, rowsB, sem_gB)
        for i in range(2):
            @pl.when(4 * (g + 1) + i < ncs)
            def _():
                _cin_issue(edge2, S, dn, e_of(4 * (g + 1) + i),
                           ei[i], sv[i], dv[i], sem_in[i])
        g2.wait()
        _conv_compute(sv[2], dv[2], w0_v, w1_v, rowsA, h)
        s2 = pltpu.async_copy(rowsA, acc.at[ei[2].at[srow]], sem_sA, add=True)
        g3.wait()
        _conv_compute(sv[3], dv[3], w0_v, w1_v, rowsB, h)
        s3 = pltpu.async_copy(rowsB, acc.at[ei[3].at[srow]], sem_sB, add=True)
        s2.wait()
        s3.wait()
        for i in range(2, 4):
            @pl.when(4 * (g + 1) + i < ncs)
            def _():
                _cin_issue(edge2, S, dn, e_of(4 * (g + 1) + i),
                           ei[i], sv[i], dv[i], sem_in[i])
        return carry

    lax.fori_loop(0, nb, body, 0)
    plsc.subcore_barrier()

    def wbody(i, carry):
        z = s + i * NT
        pltpu.sync_copy(acc.at[pl.ds(z * ZR, ZR)], out_h.at[pl.ds(z * ZR, ZR)])
        return carry

    lax.fori_loop(0, (NZ - s + NT - 1) // NT, wbody, 0)
    plsc.subcore_barrier()


def _conv_body(ue0, ue1, ie0, ie1, edge2, S, dn, zblk,
               ou0, ou1, oi0, oi1,
               acc, ei0, ei1, ei2, ei3, sv0, sv1, sv2, sv3,
               dv0, dv1, dv2, dv3, w0_v, w1_v, rowsA, rowsB,
               semi0, semi1, semi2, semi3, sem_gA, sem_gB, sem_sA, sem_sB):
    c = lax.axis_index("c")
    s = lax.axis_index("s")
    ei = (ei0, ei1, ei2, ei3)
    sv = (sv0, sv1, sv2, sv3)
    dv = (dv0, dv1, dv2, dv3)
    sem_in = (semi0, semi1, semi2, semi3)

    common = (acc, ei, sv, dv, w0_v, w1_v, rowsA, rowsB,
              sem_in, sem_gA, sem_gB, sem_sA, sem_sB, zblk, s)

    @pl.when(c == 0)
    def _():
        _conv_half(ie0, 1, 0, edge2, dn, S, ou0, 0, *common)
        _conv_half(ie1, 1, 0, edge2, dn, S, ou1, 1, *common)

    @pl.when(c == 1)
    def _():
        _conv_half(ue0, 0, 1, edge2, dn, S, oi0, 0, *common)
        _conv_half(ue1, 0, 1, edge2, dn, S, oi1, 1, *common)


_conv_call = pl.kernel(
    _conv_body,
    out_type=[
        jax.ShapeDtypeStruct((NU, H), _f32),
        jax.ShapeDtypeStruct((NU, H), _f32),
        jax.ShapeDtypeStruct((NI, H), _f32),
        jax.ShapeDtypeStruct((NI, H), _f32),
    ],
    mesh=_MESH,
    compiler_params=pltpu.CompilerParams(needs_layout_passes=False,
                                         use_tc_tiling_on_sc=False),
    scratch_types=[
        pltpu.VMEM_SHARED((NU, H), _f32),
        pltpu.VMEM((2, CH), _i32),
        pltpu.VMEM((2, CH), _i32),
        pltpu.VMEM((2, CH), _i32),
        pltpu.VMEM((2, CH), _i32),
        pltpu.VMEM((F, CH), _f32),
        pltpu.VMEM((F, CH), _f32),
        pltpu.VMEM((F, CH), _f32),
        pltpu.VMEM((F, CH), _f32),
        pltpu.VMEM((CH,), _f32),
        pltpu.VMEM((CH,), _f32),
        pltpu.VMEM((CH,), _f32),
        pltpu.VMEM((CH,), _f32),
        pltpu.VMEM((CH,), _f32),
        pltpu.VMEM((CH,), _f32),
        pltpu.VMEM((CH, H), _f32),
        pltpu.VMEM((CH, H), _f32),
        pltpu.SemaphoreType.DMA,
        pltpu.SemaphoreType.DMA,
        pltpu.SemaphoreType.DMA,
        pltpu.SemaphoreType.DMA,
        pltpu.SemaphoreType.DMA,
        pltpu.SemaphoreType.DMA,
        pltpu.SemaphoreType.DMA,
        pltpu.SemaphoreType.DMA,
    ],
)


def _sin_issue(edge2, S, e0, ei, sv, sem):
    pltpu.async_copy(edge2.at[:, pl.ds(e0, CH)], ei, sem)
    pltpu.async_copy(S.at[:, pl.ds(e0, CH)], sv, sem)


def _sin_drain(edge2, S, ei, sv, sem):
    pltpu.make_async_copy(edge2.at[:, pl.ds(0, CH)], ei, sem).wait()
    pltpu.make_async_copy(S.at[:, pl.ds(0, CH)], sv, sem).wait()


def _sval_compute(sv, sn, sv_acc, z0r, z1r, y0r, y1r):
    @plsc.parallel_loop(0, CH, unroll=4)
    def _(j):
        bj = jnp.broadcast_to(j, (16,)).astype(_i32)
        for k in range(F):
            zr, yr = (z0r, y0r) if k < 2 else (z1r, y1r)
            kk = k % 2
            za = zr[j, pl.ds(kk * 32, 16)]
            zb = zr[j, pl.ds(kk * 32 + 16, 16)]
            ya = yr[j, pl.ds(kk * 32, 16)]
            yb = yr[j, pl.ds(kk * 32 + 16, 16)]
            r = jnp.sum(za * ya + zb * yb)
            bk = jnp.full((16,), k, _i32)
            plsc.store_scatter(sv_acc, [bk, bj], jnp.broadcast_to(r, (16,)))

    @plsc.parallel_loop(0, CH // 16, unroll=2)
    def _(g):
        sl = pl.ds(g * 16, 16)
        s0 = sv[0, sl]
        s1 = sv[1, sl]
        s2 = sv[2, sl]
        s3 = sv[3, sl]
        m = jnp.maximum(jnp.maximum(s0, s1), jnp.maximum(s2, s3))
        x0 = jnp.exp(s0 - m)
        x1 = jnp.exp(s1 - m)
        x2 = jnp.exp(s2 - m)
        x3 = jnp.exp(s3 - m)
        r = jnp.full((16,), 1.0, _f32) / ((x0 + x1) + (x2 + x3))
        sn[0, sl] = x0 * r + sv_acc[0, sl]
        sn[1, sl] = x1 * r + sv_acc[1, sl]
        sn[2, sl] = x2 * r + sv_acc[2, sl]
        sn[3, sl] = x3 * r + sv_acc[3, sl]


def _sval_body(z0, z1, y0, y1, edge2, S, S_out,
               ei0, ei1, ei2, ei3, sv0, sv1, sv2, sv3, svacc,
               snA, snB, z0A, z1A, y0A, y1A, z0B, z1B, y0B, y1B,
               semi0, semi1, semi2, semi3, sem_gA, sem_gB, sem_oA, sem_oB):
    c = lax.axis_index("c")
    s = lax.axis_index("s")
    gid = c * NT + s
    ei = (ei0, ei1, ei2, ei3)
    sv = (sv0, sv1, sv2, sv3)
    sem_in = (semi0, semi1, semi2, semi3)
    ncs = NCP // NW
    nb = ncs // 4

    def e_of(n):
        return (gid + n * NW) * CH

    for i in range(4):
        _sin_issue(edge2, S, e_of(i), ei[i], sv[i], sem_in[i])

    def gfire(i, zr0, zr1, yr0, yr1, sem_g):
        _sin_drain(edge2, S, ei[i], sv[i], sem_in[i])
        pltpu.async_copy(z0.at[ei[i].at[0]], zr0, sem_g)
        pltpu.async_copy(z1.at[ei[i].at[0]], zr1, sem_g)
        pltpu.async_copy(y0.at[ei[i].at[1]], yr0, sem_g)
        pltpu.async_copy(y1.at[ei[i].at[1]], yr1, sem_g)

    def gdrain(i, zr0, zr1, yr0, yr1, sem_g):
        pltpu.make_async_copy(z0.at[ei[i].at[0]], zr0, sem_g).wait()
        pltpu.make_async_copy(z1.at[ei[i].at[0]], zr1, sem_g).wait()
        pltpu.make_async_copy(y0.at[ei[i].at[1]], yr0, sem_g).wait()
        pltpu.make_async_copy(y1.at[ei[i].at[1]], yr1, sem_g).wait()

    def body(g, carry):
        gfire(0, z0A, z1A, y0A, y1A, sem_gA)
        gfire(1, z0B, z1B, y0B, y1B, sem_gB)
        gdrain(0, z0A, z1A, y0A, y1A, sem_gA)
        _sval_compute(sv[0], snA, svacc, z0A, z1A, y0A, y1A)
        o0 = pltpu.async_copy(snA, S_out.at[:, pl.ds(e_of(4 * g), CH)], sem_oA)
        gdrain(1, z0B, z1B, y0B, y1B, sem_gB)
        _sval_compute(sv[1], snB, svacc, z0B, z1B, y0B, y1B)
        o1 = pltpu.async_copy(snB, S_out.at[:, pl.ds(e_of(4 * g + 1), CH)],
                              sem_oB)
        o0.wait()
        gfire(2, z0A, z1A, y0A, y1A, sem_gA)
        o1.wait()
        gfire(3, z0B, z1B, y0B, y1B, sem_gB)
        for i in range(2):
            @pl.when(4 * (g + 1) + i < ncs)
            def _():
                _sin_issue(edge2, S, e_of(4 * (g + 1) + i), ei[i], sv[i],
                           sem_in[i])
        gdrain(2, z0A, z1A, y0A, y1A, sem_gA)
        _sval_compute(sv[2], snA, svacc, z0A, z1A, y0A, y1A)
        o2 = pltpu.async_copy(snA, S_out.at[:, pl.ds(e_of(4 * g + 2), CH)],
                              sem_oA)
        gdrain(3, z0B, z1B, y0B, y1B, sem_gB)
        _sval_compute(sv[3], snB, svacc, z0B, z1B, y0B, y1B)
        o3 = pltpu.async_copy(snB, S_out.at[:, pl.ds(e_of(4 * g + 3), CH)],
                              sem_oB)
        o2.wait()
        o3.wait()
        for i in range(2, 4):
            @pl.when(4 * (g + 1) + i < ncs)
            def _():
                _sin_issue(edge2, S, e_of(4 * (g + 1) + i), ei[i], sv[i],
                           sem_in[i])
        return carry

    lax.fori_loop(0, nb, body, 0)


_sval_call = pl.kernel(
    _sval_body,
    out_type=[jax.ShapeDtypeStruct((F, EP), _f32)],
    mesh=_MESH,
    compiler_params=pltpu.CompilerParams(needs_layout_passes=False,
                                         use_tc_tiling_on_sc=False),
    scratch_types=[
        pltpu.VMEM((2, CH), _i32),
        pltpu.VMEM((2, CH), _i32),
        pltpu.VMEM((2, CH), _i32),
        pltpu.VMEM((2, CH), _i32),
        pltpu.VMEM((F, CH), _f32),
        pltpu.VMEM((F, CH), _f32),
        pltpu.VMEM((F, CH), _f32),
        pltpu.VMEM((F, CH), _f32),
        pltpu.VMEM((F, CH), _f32),
        pltpu.VMEM((F, CH), _f32),
        pltpu.VMEM((F, CH), _f32),
        pltpu.VMEM((CH, H), _f32),
        pltpu.VMEM((CH, H), _f32),
        pltpu.VMEM((CH, H), _f32),
        pltpu.VMEM((CH, H), _f32),
        pltpu.VMEM((CH, H), _f32),
        pltpu.VMEM((CH, H), _f32),
        pltpu.VMEM((CH, H), _f32),
        pltpu.VMEM((CH, H), _f32),
        pltpu.SemaphoreType.DMA,
        pltpu.SemaphoreType.DMA,
        pltpu.SemaphoreType.DMA,
        pltpu.SemaphoreType.DMA,
        pltpu.SemaphoreType.DMA,
        pltpu.SemaphoreType.DMA,
        pltpu.SemaphoreType.DMA,
        pltpu.SemaphoreType.DMA,
    ],
)


def _prep_body(item_ref, degu_ref, degi_ref, y0_ref, y1_ref, du_ref, di_ref):
    x = item_ref[...]
    xs = x.reshape(-1, F, DK)
    n = jnp.sqrt(jnp.sum(xs * xs, axis=2, keepdims=True))
    y = jnp.tanh((xs / jnp.maximum(n, 1e-12)).reshape(x.shape))
    y0_ref[...] = y[:, :H]
    y1_ref[...] = y[:, H:]
    for dref, oref in ((degu_ref, du_ref), (degi_ref, di_ref)):
        dg = dref[...]
        oref[...] = jnp.where(dg > 0, 1.0 / jnp.sqrt(jnp.where(dg > 0, dg, 1.0)), 0.0)


def _prep_call(item_emb, deg_u3, deg_i3):
    blk = 1000
    nb = NI // blk
    return pl.pallas_call(
        _prep_body,
        grid=(nb,),
        in_specs=[
            pl.BlockSpec((blk, D), lambda i: (i, 0)),
            pl.BlockSpec((1, 1, blk), lambda i: (i, 0, 0)),
            pl.BlockSpec((1, 1, blk), lambda i: (i, 0, 0)),
        ],
        out_specs=[
            pl.BlockSpec((blk, H), lambda i: (i, 0)),
            pl.BlockSpec((blk, H), lambda i: (i, 0)),
            pl.BlockSpec((1, 1, blk), lambda i: (i, 0, 0)),
            pl.BlockSpec((1, 1, blk), lambda i: (i, 0, 0)),
        ],
        out_shape=[
            jax.ShapeDtypeStruct((NI, H), _f32),
            jax.ShapeDtypeStruct((NI, H), _f32),
            jax.ShapeDtypeStruct((nb, 1, blk), _f32),
            jax.ShapeDtypeStruct((nb, 1, blk), _f32),
        ],
    )(item_emb, deg_u3, deg_i3)


def _znorm_body(a_ref, b_ref, z0_ref, z1_ref):
    for src, dst in ((a_ref, z0_ref), (b_ref, z1_ref)):
        x = src[...]
        xs = x.reshape(-1, 2, DK)
        n = jnp.sqrt(jnp.sum(xs * xs, axis=2, keepdims=True))
        dst[...] = (xs / jnp.maximum(n, 1e-12)).reshape(x.shape)


def _znorm_call(ou0, ou1):
    blk = 1000
    return pl.pallas_call(
        _znorm_body,
        grid=(NU // blk,),
        in_specs=[pl.BlockSpec((blk, H), lambda i: (i, 0))] * 2,
        out_specs=[pl.BlockSpec((blk, H), lambda i: (i, 0))] * 2,
        out_shape=[jax.ShapeDtypeStruct((NU, H), _f32)] * 2,
    )(ou0, ou1)


def _final_body(emb_ref, a_ref, b_ref, o_ref):
    o_ref[...] = emb_ref[...] + jnp.concatenate([a_ref[...], b_ref[...]],
                                                axis=1)


def _final_call(emb, a, b):
    blk = 1000
    return pl.pallas_call(
        _final_body,
        grid=(emb.shape[0] // blk,),
        in_specs=[
            pl.BlockSpec((blk, D), lambda i: (i, 0)),
            pl.BlockSpec((blk, H), lambda i: (i, 0)),
            pl.BlockSpec((blk, H), lambda i: (i, 0)),
        ],
        out_specs=pl.BlockSpec((blk, D), lambda i: (i, 0)),
        out_shape=jax.ShapeDtypeStruct(emb.shape, _f32),
    )(emb, a, b)


def kernel(user_emb, item_emb, edge_index, S_init):
    users = edge_index[0]
    iloc = edge_index[1] - NU
    edge2 = jnp.concatenate(
        [jnp.stack([users, iloc], axis=0),
         jnp.zeros((2, EP - E), _i32)], axis=1)
    ue0 = user_emb[:, :H]
    ue1 = user_emb[:, H:]
    ie0 = item_emb[:, :H]
    ie1 = item_emb[:, H:]
    zdeg = jnp.zeros((NU,), _f32)
    zblk = jnp.zeros((ZR, H), _f32)

    deg_u, deg_i = _deg_call(users, iloc, zdeg)
    y0, y1, du3, di3 = _prep_call(item_emb, deg_u.reshape(25, 1, 1000),
                                  deg_i.reshape(25, 1, 1000))
    dis_u = du3.reshape(NU)
    dis_i = di3.reshape(NI)
    (dn,) = _dn_call(dis_u, dis_i, edge2)

    S = jnp.concatenate([S_init, jnp.zeros((F, EP - E), _f32)], axis=1)
    ou0 = ou1 = oi0 = oi1 = None
    for _t in range(2):
        ou0, ou1, oi0, oi1 = _conv_call(ue0, ue1, ie0, ie1, edge2, S, dn, zblk)
        zu0, zu1 = _znorm_call(ou0, ou1)
        (S,) = _sval_call(zu0, zu1, y0, y1, edge2, S)

    fu = _final_call(user_emb, ou0, ou1)
    fi = _final_call(item_emb, oi0, oi1)
    return fu, fi, S[:, :E]
